# initial kernel scaffold (unmeasured)
import jax
import jax.numpy as jnp
from jax import lax
from jax.experimental import pallas as pl
from jax.experimental.pallas import tpu as pltpu


def kernel(
    x,
):
    def body(*refs):
        pass

    out_shape = jax.ShapeDtypeStruct(..., jnp.float32)
    return pl.pallas_call(body, out_shape=out_shape)(...)


# baseline (device time: 12800 ns/iter reference)
import jax
import jax.numpy as jnp
from jax import lax
from jax.experimental import pallas as pl
from jax.experimental.pallas import tpu as pltpu

N_DEV = 4


def kernel(x):
    m, n = x.shape

    def body(x_ref, out_ref, halo_ref, send_sems, recv_sems):
        my = lax.axis_index("i")
        has_left = my > 0
        has_right = my < N_DEV - 1

        barrier_sem = pltpu.get_barrier_semaphore()

        @pl.when(has_left)
        def _():
            pl.semaphore_signal(
                barrier_sem, inc=1,
                device_id=(my - 1,), device_id_type=pl.DeviceIdType.MESH,
            )

        @pl.when(has_right)
        def _():
            pl.semaphore_signal(
                barrier_sem, inc=1,
                device_id=(my + 1,), device_id_type=pl.DeviceIdType.MESH,
            )

        @pl.when(has_left)
        def _():
            pl.semaphore_wait(barrier_sem, 1)

        @pl.when(has_right)
        def _():
            pl.semaphore_wait(barrier_sem, 1)

        @pl.when(has_right)
        def _():
            rdma = pltpu.make_async_remote_copy(
                src_ref=x_ref.at[pl.ds(m - 1, 1)],
                dst_ref=halo_ref.at[0],
                send_sem=send_sems.at[0],
                recv_sem=recv_sems.at[0],
                device_id=(my + 1,),
                device_id_type=pl.DeviceIdType.MESH,
            )
            rdma.start()

        @pl.when(has_left)
        def _():
            rdma = pltpu.make_async_remote_copy(
                src_ref=x_ref.at[pl.ds(0, 1)],
                dst_ref=halo_ref.at[1],
                send_sem=send_sems.at[1],
                recv_sem=recv_sems.at[1],
                device_id=(my - 1,),
                device_id_type=pl.DeviceIdType.MESH,
            )
            rdma.start()

        @pl.when(has_left)
        def _():
            recv = pltpu.make_async_remote_copy(
                src_ref=x_ref.at[pl.ds(0, 1)],
                dst_ref=halo_ref.at[0],
                send_sem=send_sems.at[0],
                recv_sem=recv_sems.at[0],
                device_id=(my - 1,),
                device_id_type=pl.DeviceIdType.MESH,
            )
            recv.wait_recv()

        @pl.when(has_right)
        def _():
            recv = pltpu.make_async_remote_copy(
                src_ref=x_ref.at[pl.ds(m - 1, 1)],
                dst_ref=halo_ref.at[1],
                send_sem=send_sems.at[1],
                recv_sem=recv_sems.at[1],
                device_id=(my + 1,),
                device_id_type=pl.DeviceIdType.MESH,
            )
            recv.wait_recv()

        x_val = x_ref[:, :]
        top = halo_ref[0]
        bot = halo_ref[1]
        up = jnp.concatenate([top, x_val[: m - 1, :]], axis=0)
        down = jnp.concatenate([x_val[1:, :], bot], axis=0)
        out = 0.25 * up + 0.5 * x_val + 0.25 * down
        out_ref[:, :] = out.astype(out_ref.dtype)

        @pl.when(my == 0)
        def _():
            out_ref[0:1, :] = x_ref[0:1, :].astype(out_ref.dtype)

        @pl.when(my == N_DEV - 1)
        def _():
            out_ref[m - 1 : m, :] = x_ref[m - 1 : m, :].astype(out_ref.dtype)

        @pl.when(has_right)
        def _():
            send = pltpu.make_async_remote_copy(
                src_ref=x_ref.at[pl.ds(m - 1, 1)],
                dst_ref=halo_ref.at[0],
                send_sem=send_sems.at[0],
                recv_sem=recv_sems.at[0],
                device_id=(my + 1,),
                device_id_type=pl.DeviceIdType.MESH,
            )
            send.wait_send()

        @pl.when(has_left)
        def _():
            send = pltpu.make_async_remote_copy(
                src_ref=x_ref.at[pl.ds(0, 1)],
                dst_ref=halo_ref.at[1],
                send_sem=send_sems.at[1],
                recv_sem=recv_sems.at[1],
                device_id=(my - 1,),
                device_id_type=pl.DeviceIdType.MESH,
            )
            send.wait_send()

    return pl.pallas_call(
        body,
        out_shape=jax.ShapeDtypeStruct((m, n), jnp.bfloat16),
        in_specs=[pl.BlockSpec(memory_space=pltpu.VMEM)],
        out_specs=pl.BlockSpec(memory_space=pltpu.VMEM),
        scratch_shapes=[
            pltpu.VMEM((2, 1, n), x.dtype),
            pltpu.SemaphoreType.DMA((2,)),
            pltpu.SemaphoreType.DMA((2,)),
        ],
        compiler_params=pltpu.CompilerParams(collective_id=0),
    )(x)


# device time: 10500 ns/iter; 1.2190x vs baseline; 1.2190x over previous
import jax
import jax.numpy as jnp
from jax import lax
from jax.experimental import pallas as pl
from jax.experimental.pallas import tpu as pltpu

N_DEV = 4
NB = 8
PAD = 8


def kernel(x):
    m, n = x.shape
    B = m // NB
    order = list(range(1, NB - 1)) + [0, NB - 1]

    def body(
        x_hbm,
        out_hbm,
        in_buf,
        out_buf,
        halo_ref,
        stage,
        in_sems,
        out_sems,
        stage_sem,
        send_sems,
        recv_sems,
    ):
        my = lax.axis_index("i")
        has_left = my > 0
        has_right = my < N_DEV - 1

        def make_in(b, slot):
            if b == 0:
                return pltpu.make_async_copy(
                    x_hbm.at[pl.ds(0, B + PAD)],
                    in_buf.at[slot, pl.ds(PAD, B + PAD)],
                    in_sems.at[slot],
                )
            if b == NB - 1:
                return pltpu.make_async_copy(
                    x_hbm.at[pl.ds(m - B - PAD, B + PAD)],
                    in_buf.at[slot, pl.ds(0, B + PAD)],
                    in_sems.at[slot],
                )
            return pltpu.make_async_copy(
                x_hbm.at[pl.ds(b * B - PAD, B + 2 * PAD)],
                in_buf.at[slot, pl.ds(0, B + 2 * PAD)],
                in_sems.at[slot],
            )

        def make_out(b, slot):
            return pltpu.make_async_copy(
                out_buf.at[slot],
                out_hbm.at[pl.ds(b * B, B)],
                out_sems.at[slot],
            )

        stage_cps = [
            pltpu.make_async_copy(
                x_hbm.at[pl.ds(0, PAD)], stage.at[0], stage_sem.at[0]
            ),
            pltpu.make_async_copy(
                x_hbm.at[pl.ds(m - PAD, PAD)], stage.at[1], stage_sem.at[0]
            ),
        ]
        for c in stage_cps:
            c.start()
        ins = [make_in(order[i], i % 2) for i in range(NB)]
        ins[0].start()

        barrier_sem = pltpu.get_barrier_semaphore()

        @pl.when(has_left)
        def _():
            pl.semaphore_signal(
                barrier_sem, inc=1,
                device_id=(my - 1,), device_id_type=pl.DeviceIdType.MESH,
            )

        @pl.when(has_right)
        def _():
            pl.semaphore_signal(
                barrier_sem, inc=1,
                device_id=(my + 1,), device_id_type=pl.DeviceIdType.MESH,
            )

        @pl.when(has_left)
        def _():
            pl.semaphore_wait(barrier_sem, 1)

        @pl.when(has_right)
        def _():
            pl.semaphore_wait(barrier_sem, 1)

        for c in stage_cps:
            c.wait()

        @pl.when(has_right)
        def _():
            pltpu.make_async_remote_copy(
                src_ref=stage.at[1, pl.ds(PAD - 1, 1)],
                dst_ref=halo_ref.at[0],
                send_sem=send_sems.at[0],
                recv_sem=recv_sems.at[0],
                device_id=(my + 1,),
                device_id_type=pl.DeviceIdType.MESH,
            ).start()

        @pl.when(has_left)
        def _():
            pltpu.make_async_remote_copy(
                src_ref=stage.at[0, pl.ds(0, 1)],
                dst_ref=halo_ref.at[1],
                send_sem=send_sems.at[1],
                recv_sem=recv_sems.at[1],
                device_id=(my - 1,),
                device_id_type=pl.DeviceIdType.MESH,
            ).start()

        outs = [None] * NB
        for idx in range(NB):
            b = order[idx]
            slot = idx % 2
            ins[idx].wait()
            if idx + 1 < NB:
                ins[idx + 1].start()
            if idx >= 2:
                outs[idx - 2].wait()

            if b == 0:
                @pl.when(has_left)
                def _():
                    pltpu.make_async_remote_copy(
                        src_ref=x_hbm.at[pl.ds(0, 1)],
                        dst_ref=halo_ref.at[0],
                        send_sem=send_sems.at[0],
                        recv_sem=recv_sems.at[0],
                        device_id=(my - 1,),
                        device_id_type=pl.DeviceIdType.MESH,
                    ).wait_recv()
                in_buf[slot, PAD - 1 : PAD, :] = halo_ref[0]
            if b == NB - 1:
                @pl.when(has_right)
                def _():
                    pltpu.make_async_remote_copy(
                        src_ref=x_hbm.at[pl.ds(m - 1, 1)],
                        dst_ref=halo_ref.at[1],
                        send_sem=send_sems.at[1],
                        recv_sem=recv_sems.at[1],
                        device_id=(my + 1,),
                        device_id_type=pl.DeviceIdType.MESH,
                    ).wait_recv()
                in_buf[slot, PAD + B : PAD + B + 1, :] = halo_ref[1]

            up = in_buf[slot, PAD - 1 : PAD - 1 + B, :]
            c0 = in_buf[slot, PAD : PAD + B, :]
            down = in_buf[slot, PAD + 1 : PAD + 1 + B, :]
            out = (up + down) * 0.25 + c0 * 0.5
            out_buf[slot, :, :] = out.astype(out_buf.dtype)

            if b == 0:
                @pl.when(my == 0)
                def _():
                    out_buf[slot, 0:1, :] = in_buf[slot, PAD : PAD + 1, :].astype(
                        out_buf.dtype
                    )
            if b == NB - 1:
                @pl.when(my == N_DEV - 1)
                def _():
                    out_buf[slot, B - 1 : B, :] = in_buf[
                        slot, PAD + B - 1 : PAD + B, :
                    ].astype(out_buf.dtype)

            outs[idx] = make_out(b, slot)
            outs[idx].start()

        outs[NB - 2].wait()
        outs[NB - 1].wait()

        @pl.when(has_right)
        def _():
            pltpu.make_async_remote_copy(
                src_ref=stage.at[1, pl.ds(PAD - 1, 1)],
                dst_ref=halo_ref.at[0],
                send_sem=send_sems.at[0],
                recv_sem=recv_sems.at[0],
                device_id=(my + 1,),
                device_id_type=pl.DeviceIdType.MESH,
            ).wait_send()

        @pl.when(has_left)
        def _():
            pltpu.make_async_remote_copy(
                src_ref=stage.at[0, pl.ds(0, 1)],
                dst_ref=halo_ref.at[1],
                send_sem=send_sems.at[1],
                recv_sem=recv_sems.at[1],
                device_id=(my - 1,),
                device_id_type=pl.DeviceIdType.MESH,
            ).wait_send()

    return pl.pallas_call(
        body,
        out_shape=jax.ShapeDtypeStruct((m, n), jnp.bfloat16),
        in_specs=[pl.BlockSpec(memory_space=pltpu.MemorySpace.HBM)],
        out_specs=pl.BlockSpec(memory_space=pltpu.MemorySpace.HBM),
        scratch_shapes=[
            pltpu.VMEM((2, B + 2 * PAD, n), x.dtype),
            pltpu.VMEM((2, B, n), jnp.bfloat16),
            pltpu.VMEM((2, 1, n), x.dtype),
            pltpu.VMEM((2, PAD, n), x.dtype),
            pltpu.SemaphoreType.DMA((2,)),
            pltpu.SemaphoreType.DMA((2,)),
            pltpu.SemaphoreType.DMA((1,)),
            pltpu.SemaphoreType.DMA((2,)),
            pltpu.SemaphoreType.DMA((2,)),
        ],
        compiler_params=pltpu.CompilerParams(collective_id=0),
    )(x)
